# baseline (speedup 1.0000x reference)
import dataclasses
import functools

import jax
import jax.numpy as jnp
from jax import lax
from jax.experimental import pallas as pl
from jax.experimental.pallas import tpu as pltpu
from jax.experimental.pallas import tpu_sc as plsc

N = 10000
E = 320000
NC = 2
NS = 16
NW = NC * NS
K = 128
C = 80
EPT = C * K
TOT = NW * EPT
RPT = 632
NR = NS * RPT
_ZCH = [64] * 9 + [56]

def _mesh():
    return plsc.VectorSubcoreMesh(core_axis_name="c", subcore_axis_name="s")


def _no_layout_params():
    cp = pltpu.CompilerParams()
    if "needs_layout_passes" in pltpu.CompilerParams.__dataclass_fields__:
        cp = dataclasses.replace(cp, needs_layout_passes=False)
    return cp


def _deg_pass(srcp, dstp):
    @functools.partial(
        pl.kernel,
        out_type=(jax.ShapeDtypeStruct((NW * NR,), jnp.float32),
                  jax.ShapeDtypeStruct((NW * NR,), jnp.float32)),
        mesh=_mesh(),
        compiler_params=_no_layout_params(),
        scratch_types=[
            pltpu.VMEM((C, K), jnp.int32),
            pltpu.VMEM((C, K), jnp.int32),
            pltpu.VMEM((NR,), jnp.float32),
            pltpu.VMEM((NR,), jnp.float32),
        ],
    )
    def k(srcp_hbm, dstp_hbm, dego_hbm, degi_hbm,
          src_v, dst_v, do_v, di_v):
        core = lax.axis_index("c")
        sub = lax.axis_index("s")
        w = core * NS + sub
        pltpu.sync_copy(srcp_hbm.at[w], src_v)
        pltpu.sync_copy(dstp_hbm.at[w], dst_v)

        zeros16 = jnp.zeros((16,), jnp.float32)

        @pl.loop(0, NR, step=16)
        def _(r):
            do_v[pl.ds(r, 16)] = zeros16
            di_v[pl.ds(r, 16)] = zeros16

        ones16 = jnp.ones((16,), jnp.float32)

        @pl.loop(0, C)
        def _(j):
            for cc in range(K // 16):
                plsc.addupdate_scatter(
                    do_v, [src_v[j, pl.ds(cc * 16, 16)]], ones16)
                plsc.addupdate_scatter(
                    di_v, [dst_v[j, pl.ds(cc * 16, 16)]], ones16)

        pltpu.sync_copy(do_v, dego_hbm.at[pl.ds(w * NR, NR)])
        pltpu.sync_copy(di_v, degi_hbm.at[pl.ds(w * NR, NR)])

    return k(srcp, dstp)


K2 = 32
C2 = 320


def _agg_pass(table, srcp, dstp2, d, nb=8):
    @functools.partial(
        pl.kernel,
        out_type=jax.ShapeDtypeStruct((NC, NR, d), jnp.float32),
        mesh=_mesh(),
        scratch_types=[
            pltpu.VMEM((C, K), jnp.int32),
            pltpu.VMEM((nb, K2), jnp.int32),
        ] + [pltpu.VMEM((K2, d), jnp.float32)] * nb
          + [pltpu.VMEM_SHARED((NR, d), jnp.float32)]
          + [pltpu.SemaphoreType.DMA] * (3 * nb),
    )
    def k(table_hbm, srcp_hbm, dstp2_hbm, out_hbm, src_v, dslot, *rest):
        bufs = rest[:nb]
        acc_sh = rest[nb]
        gs = rest[nb + 1:2 * nb + 1]
        isem = rest[2 * nb + 1:3 * nb + 1]
        ss = rest[3 * nb + 1:4 * nb + 1]
        core = lax.axis_index("c")
        sub = lax.axis_index("s")
        w = core * NS + sub

        def sidx(j):
            return src_v.at[j // (K // K2), pl.ds((j % (K // K2)) * K2, K2)]

        @pl.loop(0, K2)
        def _(r):
            for cc in range(d // 16):
                bufs[0][r, pl.ds(cc * 16, 16)] = jnp.zeros((16,), jnp.float32)

        r0 = sub * RPT
        off = 0
        for ch in _ZCH:
            pltpu.sync_copy(bufs[0].at[pl.ds(0, ch)],
                            acc_sh.at[pl.ds(r0 + off, ch)])
            off += ch
        plsc.subcore_barrier()

        pltpu.sync_copy(srcp_hbm.at[w], src_v)
        for b in range(nb):
            pltpu.async_copy(dstp2_hbm.at[w, b], dslot.at[b], isem[b])
            pltpu.async_copy(table_hbm.at[sidx(b)], bufs[b], gs[b])

        @pl.loop(0, C2 // nb - 1)
        def _(i):
            base = i * nb
            for b in range(nb):
                j = base + b
                pltpu.make_async_copy(
                    table_hbm.at[sidx(j)], bufs[b], gs[b]).wait()
                pltpu.make_async_copy(
                    dstp2_hbm.at[w, j], dslot.at[b], isem[b]).wait()
                pltpu.async_copy(bufs[b], acc_sh.at[dslot.at[b]], ss[b],
                                 add=True)
            for b in range(nb):
                j = base + b
                pltpu.make_async_copy(
                    bufs[b], acc_sh.at[dslot.at[b]], ss[b]).wait()
                pltpu.async_copy(dstp2_hbm.at[w, j + nb], dslot.at[b],
                                 isem[b])
                pltpu.async_copy(table_hbm.at[sidx(j + nb)], bufs[b], gs[b])

        for b in range(nb):
            j = C2 - nb + b
            pltpu.make_async_copy(
                table_hbm.at[sidx(j)], bufs[b], gs[b]).wait()
            pltpu.make_async_copy(
                dstp2_hbm.at[w, j], dslot.at[b], isem[b]).wait()
            pltpu.async_copy(bufs[b], acc_sh.at[dslot.at[b]], ss[b],
                             add=True)
        for b in range(nb):
            pltpu.make_async_copy(
                bufs[b], acc_sh.at[dslot.at[b]], ss[b]).wait()

        plsc.subcore_barrier()
        pltpu.sync_copy(acc_sh.at[pl.ds(r0, RPT)],
                        out_hbm.at[core, pl.ds(r0, RPT)])

    return k(table, srcp, dstp2)


def _scale_pass(x, dego, degi):
    def body(x_ref, do_ref, di_ref, xs_ref, ns_ref, nd_ref):
        do = jnp.sum(do_ref[...], axis=1, keepdims=True)
        di = jnp.sum(di_ref[...], axis=1, keepdims=True)
        ns = jnp.where(do > 0, lax.rsqrt(jnp.maximum(do, 1.0)), 0.0)
        nd = jnp.where(di > 0, lax.rsqrt(jnp.maximum(di, 1.0)), 0.0)
        ns_ref[...] = ns
        nd_ref[...] = nd
        xs_ref[...] = x_ref[...] * ns

    return pl.pallas_call(
        body,
        out_shape=(jax.ShapeDtypeStruct((N, 128), jnp.float32),
                   jax.ShapeDtypeStruct((N, 1), jnp.float32),
                   jax.ShapeDtypeStruct((N, 1), jnp.float32)),
    )(x, dego, degi)


_RB = 2000


def _mm_pass(aggp, nd, ns, w1, b1, w2):
    def body(ap_ref, nd_ref, ns_ref, w1_ref, b1_ref, w2_ref, o_ref):
        a = (ap_ref[0] + ap_ref[1]) * nd_ref[...]
        h = jnp.dot(a, w1_ref[...], preferred_element_type=jnp.float32)
        h = jnp.maximum(h + b1_ref[...], 0.0) * ns_ref[...]
        o_ref[...] = jnp.dot(h, w2_ref[...], preferred_element_type=jnp.float32)

    return pl.pallas_call(
        body,
        grid=(N // _RB,),
        in_specs=[
            pl.BlockSpec((2, _RB, 128), lambda i: (0, i, 0)),
            pl.BlockSpec((_RB, 1), lambda i: (i, 0)),
            pl.BlockSpec((_RB, 1), lambda i: (i, 0)),
            pl.BlockSpec((128, 256), lambda i: (0, 0)),
            pl.BlockSpec((1, 256), lambda i: (0, 0)),
            pl.BlockSpec((256, 64), lambda i: (0, 0)),
        ],
        out_specs=pl.BlockSpec((_RB, 64), lambda i: (i, 0)),
        out_shape=jax.ShapeDtypeStruct((N, 64), jnp.float32),
    )(aggp, nd, ns, w1, b1, w2)


def _final_pass(aggp2, nd, b2):
    def body(ap_ref, nd_ref, b2_ref, o_ref):
        o_ref[...] = jnp.maximum(
            (ap_ref[0] + ap_ref[1]) * nd_ref[...] + b2_ref[...], 0.0)

    return pl.pallas_call(
        body,
        out_shape=jax.ShapeDtypeStruct((N, 64), jnp.float32),
    )(aggp2, nd, b2)


def kernel(x, edge_index, W1, b1, W2, b2):
    src = edge_index[0].astype(jnp.int32)
    dst = edge_index[1].astype(jnp.int32)
    pad_src = jnp.full((TOT - E,), N, jnp.int32)
    pad_dst = N + (jnp.arange(TOT - E, dtype=jnp.int32) % (NR - N))
    srcp = jnp.concatenate([src, pad_src]).reshape(NW, C, K)
    dstp = jnp.concatenate([dst, pad_dst]).reshape(NW, C, K)

    degof, degif = _deg_pass(srcp, dstp)
    dego = degof.reshape(NW, NR)[:, :N].T
    degi = degif.reshape(NW, NR)[:, :N].T
    xs, ns, nd = _scale_pass(x, dego, degi)

    dstp2 = dstp.reshape(NW, C2, K2)
    zero128 = jnp.zeros((1, 128), jnp.float32)
    aggp1 = _agg_pass(jnp.concatenate([xs, zero128], axis=0),
                      srcp, dstp2, 128)[:, :N]
    m2 = _mm_pass(aggp1, nd, ns, W1, jnp.reshape(b1, (1, 256)), W2)
    m2p = jnp.concatenate(
        [m2, jnp.zeros((N, 64), jnp.float32)], axis=1)
    m2p = jnp.concatenate([m2p, jnp.zeros((1, 128), jnp.float32)], axis=0)
    aggp2 = _agg_pass(m2p, srcp, dstp2, 128)[:, :N, :64]
    return _final_pass(aggp2, nd, jnp.reshape(b2, (1, 64)))


# speedup vs baseline: 1.1688x; 1.1688x over previous
import dataclasses
import functools

import jax
import jax.numpy as jnp
from jax import lax
from jax.experimental import pallas as pl
from jax.experimental.pallas import tpu as pltpu
from jax.experimental.pallas import tpu_sc as plsc

N = 10000
E = 320000
NC = 2
NS = 16
NW = NC * NS
K = 128
C = 80
EPT = C * K
TOT = NW * EPT
RPT = 632
NR = NS * RPT
_ZCH = [64] * 9 + [56]

def _mesh():
    return plsc.VectorSubcoreMesh(core_axis_name="c", subcore_axis_name="s")


def _no_layout_params():
    cp = pltpu.CompilerParams()
    if "needs_layout_passes" in pltpu.CompilerParams.__dataclass_fields__:
        cp = dataclasses.replace(cp, needs_layout_passes=False)
    return cp


def _deg_pass(srcp, dstp):
    @functools.partial(
        pl.kernel,
        out_type=(jax.ShapeDtypeStruct((NW * NR,), jnp.float32),
                  jax.ShapeDtypeStruct((NW * NR,), jnp.float32)),
        mesh=_mesh(),
        compiler_params=_no_layout_params(),
        scratch_types=[
            pltpu.VMEM((C, K), jnp.int32),
            pltpu.VMEM((C, K), jnp.int32),
            pltpu.VMEM((NR,), jnp.float32),
            pltpu.VMEM((NR,), jnp.float32),
        ],
    )
    def k(srcp_hbm, dstp_hbm, dego_hbm, degi_hbm,
          src_v, dst_v, do_v, di_v):
        core = lax.axis_index("c")
        sub = lax.axis_index("s")
        w = core * NS + sub
        pltpu.sync_copy(srcp_hbm.at[w], src_v)
        pltpu.sync_copy(dstp_hbm.at[w], dst_v)

        zeros16 = jnp.zeros((16,), jnp.float32)

        @pl.loop(0, NR, step=16)
        def _(r):
            do_v[pl.ds(r, 16)] = zeros16
            di_v[pl.ds(r, 16)] = zeros16

        ones16 = jnp.ones((16,), jnp.float32)

        @pl.loop(0, C)
        def _(j):
            for cc in range(K // 16):
                plsc.addupdate_scatter(
                    do_v, [src_v[j, pl.ds(cc * 16, 16)]], ones16)
                plsc.addupdate_scatter(
                    di_v, [dst_v[j, pl.ds(cc * 16, 16)]], ones16)

        pltpu.sync_copy(do_v, dego_hbm.at[pl.ds(w * NR, NR)])
        pltpu.sync_copy(di_v, degi_hbm.at[pl.ds(w * NR, NR)])

    return k(srcp, dstp)


K2 = 64
C2 = 160


def _agg_pass(table, srcp, dstp2, d, nb=4):
    @functools.partial(
        pl.kernel,
        out_type=jax.ShapeDtypeStruct((NC, NR, d), jnp.float32),
        mesh=_mesh(),
        scratch_types=[
            pltpu.VMEM((C, K), jnp.int32),
            pltpu.VMEM((nb, K2), jnp.int32),
        ] + [pltpu.VMEM((K2, d), jnp.float32)] * nb
          + [pltpu.VMEM_SHARED((NR, d), jnp.float32)]
          + [pltpu.SemaphoreType.DMA] * (3 * nb),
    )
    def k(table_hbm, srcp_hbm, dstp2_hbm, out_hbm, src_v, dslot, *rest):
        bufs = rest[:nb]
        acc_sh = rest[nb]
        gs = rest[nb + 1:2 * nb + 1]
        isem = rest[2 * nb + 1:3 * nb + 1]
        ss = rest[3 * nb + 1:4 * nb + 1]
        core = lax.axis_index("c")
        sub = lax.axis_index("s")
        w = core * NS + sub

        def sidx(j):
            return src_v.at[j // 2, pl.ds((j % 2) * K2, K2)]

        @pl.loop(0, K2)
        def _(r):
            for cc in range(d // 16):
                bufs[0][r, pl.ds(cc * 16, 16)] = jnp.zeros((16,), jnp.float32)

        r0 = sub * RPT
        off = 0
        for ch in _ZCH:
            pltpu.sync_copy(bufs[0].at[pl.ds(0, ch)],
                            acc_sh.at[pl.ds(r0 + off, ch)])
            off += ch
        plsc.subcore_barrier()

        pltpu.sync_copy(srcp_hbm.at[w], src_v)
        for b in range(nb):
            pltpu.async_copy(dstp2_hbm.at[w, b], dslot.at[b], isem[b])
            pltpu.async_copy(table_hbm.at[sidx(b)], bufs[b], gs[b])

        @pl.loop(0, C2 // nb - 1)
        def _(i):
            base = i * nb
            for b in range(nb):
                j = base + b
                pltpu.make_async_copy(
                    table_hbm.at[sidx(j)], bufs[b], gs[b]).wait()
                pltpu.make_async_copy(
                    dstp2_hbm.at[w, j], dslot.at[b], isem[b]).wait()
                pltpu.async_copy(bufs[b], acc_sh.at[dslot.at[b]], ss[b],
                                 add=True)
            for b in range(nb):
                j = base + b
                pltpu.make_async_copy(
                    bufs[b], acc_sh.at[dslot.at[b]], ss[b]).wait()
                pltpu.async_copy(dstp2_hbm.at[w, j + nb], dslot.at[b],
                                 isem[b])
                pltpu.async_copy(table_hbm.at[sidx(j + nb)], bufs[b], gs[b])

        for b in range(nb):
            j = C2 - nb + b
            pltpu.make_async_copy(
                table_hbm.at[sidx(j)], bufs[b], gs[b]).wait()
            pltpu.make_async_copy(
                dstp2_hbm.at[w, j], dslot.at[b], isem[b]).wait()
            pltpu.async_copy(bufs[b], acc_sh.at[dslot.at[b]], ss[b],
                             add=True)
        for b in range(nb):
            pltpu.make_async_copy(
                bufs[b], acc_sh.at[dslot.at[b]], ss[b]).wait()

        plsc.subcore_barrier()
        pltpu.sync_copy(acc_sh.at[pl.ds(r0, RPT)],
                        out_hbm.at[core, pl.ds(r0, RPT)])

    return k(table, srcp, dstp2)


def _scale_pass(x, dego, degi):
    def body(x_ref, do_ref, di_ref, xs_ref, ns_ref, nd_ref):
        do = jnp.sum(do_ref[...], axis=1, keepdims=True)
        di = jnp.sum(di_ref[...], axis=1, keepdims=True)
        ns = jnp.where(do > 0, lax.rsqrt(jnp.maximum(do, 1.0)), 0.0)
        nd = jnp.where(di > 0, lax.rsqrt(jnp.maximum(di, 1.0)), 0.0)
        ns_ref[...] = ns
        nd_ref[...] = nd
        xs_ref[...] = x_ref[...] * ns

    return pl.pallas_call(
        body,
        out_shape=(jax.ShapeDtypeStruct((N, 128), jnp.float32),
                   jax.ShapeDtypeStruct((N, 1), jnp.float32),
                   jax.ShapeDtypeStruct((N, 1), jnp.float32)),
    )(x, dego, degi)


_RB = 2000


def _mm_pass(aggp, nd, ns, w1, b1, w2):
    def body(ap_ref, nd_ref, ns_ref, w1_ref, b1_ref, w2_ref, o_ref):
        a = (ap_ref[0] + ap_ref[1]) * nd_ref[...]
        h = jnp.dot(a, w1_ref[...], preferred_element_type=jnp.float32)
        h = jnp.maximum(h + b1_ref[...], 0.0) * ns_ref[...]
        o_ref[...] = jnp.dot(h, w2_ref[...], preferred_element_type=jnp.float32)

    return pl.pallas_call(
        body,
        grid=(N // _RB,),
        in_specs=[
            pl.BlockSpec((2, _RB, 128), lambda i: (0, i, 0)),
            pl.BlockSpec((_RB, 1), lambda i: (i, 0)),
            pl.BlockSpec((_RB, 1), lambda i: (i, 0)),
            pl.BlockSpec((128, 256), lambda i: (0, 0)),
            pl.BlockSpec((1, 256), lambda i: (0, 0)),
            pl.BlockSpec((256, 64), lambda i: (0, 0)),
        ],
        out_specs=pl.BlockSpec((_RB, 64), lambda i: (i, 0)),
        out_shape=jax.ShapeDtypeStruct((N, 64), jnp.float32),
    )(aggp, nd, ns, w1, b1, w2)


def _final_pass(aggp2, nd, b2):
    def body(ap_ref, nd_ref, b2_ref, o_ref):
        o_ref[...] = jnp.maximum(
            (ap_ref[0] + ap_ref[1]) * nd_ref[...] + b2_ref[...], 0.0)

    return pl.pallas_call(
        body,
        out_shape=jax.ShapeDtypeStruct((N, 64), jnp.float32),
    )(aggp2, nd, b2)


def kernel(x, edge_index, W1, b1, W2, b2):
    src = edge_index[0].astype(jnp.int32)
    dst = edge_index[1].astype(jnp.int32)
    pad_src = jnp.full((TOT - E,), N, jnp.int32)
    pad_dst = N + (jnp.arange(TOT - E, dtype=jnp.int32) % (NR - N))
    srcp = jnp.concatenate([src, pad_src]).reshape(NW, C, K)
    dstp = jnp.concatenate([dst, pad_dst]).reshape(NW, C, K)

    degof, degif = _deg_pass(srcp, dstp)
    dego = degof.reshape(NW, NR)[:, :N].T
    degi = degif.reshape(NW, NR)[:, :N].T
    xs, ns, nd = _scale_pass(x, dego, degi)

    dstp2 = dstp.reshape(NW, C2, K2)
    zero128 = jnp.zeros((1, 128), jnp.float32)
    aggp1 = _agg_pass(jnp.concatenate([xs, zero128], axis=0),
                      srcp, dstp2, 128)[:, :N]
    m2 = _mm_pass(aggp1, nd, ns, W1, jnp.reshape(b1, (1, 256)), W2)
    m2p = jnp.concatenate(
        [m2, jnp.zeros((N, 64), jnp.float32)], axis=1)
    m2p = jnp.concatenate([m2p, jnp.zeros((1, 128), jnp.float32)], axis=0)
    aggp2 = _agg_pass(m2p, srcp, dstp2, 128)[:, :N, :64]
    return _final_pass(aggp2, nd, jnp.reshape(b2, (1, 64)))
